# baseline (speedup 1.0000x reference)
import functools

import numpy as np
import jax
import jax.numpy as jnp
from jax import lax
from jax.experimental import pallas as pl
from jax.experimental.pallas import tpu as pltpu
from jax.experimental.pallas import tpu_sc as plsc

_FIELDS = 26
_FIELD_DIM = 100000
_EMBED = 32
_BATCH = 16384
_N = _BATCH * _FIELDS
_NC, _NS, _L = 2, 16, 16
_NW = _NC * _NS
_NPW = _N // _NW
_CHUNK = 128
_NCHUNK = _NPW // _CHUNK
_K = 8
_NGRP = _NCHUNK // _K

_PATTERN = np.asarray(
    (np.arange(_NPW, dtype=np.int64) % _FIELDS) * _FIELD_DIM, dtype=np.int32
)

_mesh = plsc.VectorSubcoreMesh(core_axis_name="c", subcore_axis_name="s")


@functools.partial(
    pl.kernel,
    mesh=_mesh,
    out_type=jax.ShapeDtypeStruct((_N, _EMBED), jnp.float32),
    scratch_types=[
        pltpu.VMEM((_NPW,), jnp.int32),
        pltpu.VMEM((_NPW,), jnp.int32),
        pltpu.VMEM((2, _K, _CHUNK, _EMBED), jnp.float32),
        pltpu.SemaphoreType.DMA,
        pltpu.SemaphoreType.DMA,
    ],
    compiler_params=pltpu.CompilerParams(use_tc_tiling_on_sc=False),
)
def _embedding_gather(x_hbm, pat_hbm, table_hbm, out_hbm,
                      idx_v, pat_v, bufs, gsem, wsem):
    wid = lax.axis_index("s") * _NC + lax.axis_index("c")
    base = wid * _NPW

    pltpu.sync_copy(x_hbm.at[pl.ds(base, _NPW)], idx_v)
    pltpu.sync_copy(pat_hbm, pat_v)

    def _add_offsets(i, carry):
        s = i * _L
        idx_v[pl.ds(s, _L)] = idx_v[pl.ds(s, _L)] + pat_v[pl.ds(s, _L)]
        return carry

    lax.fori_loop(0, _NPW // _L, _add_offsets, 0)

    pending_writes = []
    for grp in range(_NGRP):
        par = grp % 2
        gathers = []
        for j in range(_K):
            g = grp * _K + j
            gathers.append(
                pltpu.async_copy(
                    table_hbm.at[idx_v.at[pl.ds(g * _CHUNK, _CHUNK)]],
                    bufs.at[par, j],
                    gsem,
                )
            )
        for w in pending_writes:
            w.wait()
        pending_writes = []
        for cp in gathers:
            cp.wait()
        for j in range(_K):
            g = grp * _K + j
            pending_writes.append(
                pltpu.async_copy(
                    bufs.at[par, j],
                    out_hbm.at[pl.ds(base + g * _CHUNK, _CHUNK)],
                    wsem,
                )
            )
    for w in pending_writes:
        w.wait()


def kernel(x, table):
    xflat = x.reshape(_N)
    pat = jnp.asarray(_PATTERN)
    out = _embedding_gather(xflat, pat, table)
    return out.reshape(_BATCH, _FIELDS, _EMBED)
